# baseline (speedup 1.0000x reference)
import jax
import jax.numpy as jnp
from jax import lax
from jax.experimental import pallas as pl
from jax.experimental.pallas import tpu as pltpu
from jax.experimental.pallas import tpu_sc as plsc

CMAX = 1024
NBUF = 5
LA = 3


def _sc_gather_body(labels_hbm, emb_hbm, out_hbm, idx_v, rows_v, sem):
    wid = lax.axis_index("s") * 2 + lax.axis_index("c")

    @pl.when(wid == 0)
    def _():
        pltpu.sync_copy(labels_hbm, idx_v)
        pltpu.async_copy(emb_hbm.at[idx_v], rows_v, sem).wait()
        pltpu.sync_copy(rows_v, out_hbm)


def _sc_gather(labels8, emb_weight):
    d = emb_weight.shape[1]
    mesh = plsc.VectorSubcoreMesh(core_axis_name="c", subcore_axis_name="s")
    return pl.kernel(
        _sc_gather_body,
        mesh=mesh,
        out_type=jax.ShapeDtypeStruct((8, d), jnp.float32),
        scratch_types=[
            pltpu.VMEM((8,), jnp.int32),
            pltpu.VMEM((8, d), jnp.float32),
            pltpu.SemaphoreType.DMA,
        ],
    )(labels8, emb_weight)


def _chunk_schedule(batch, seq):
    first = [256, 256, 512] + [CMAX] * ((seq - 1024) // CMAX)
    mid = [CMAX] * (seq // CMAX)
    last = list(reversed(first))
    chunks = []
    for b in range(batch):
        sizes = first if b == 0 else (last if b == batch - 1 else mid)
        r = b * seq
        for sz in sizes:
            chunks.append((r, sz, b))
            r += sz
    return chunks


def _make_body(batch, seq, d):
    chunks = _chunk_schedule(batch, seq)
    nchunk = len(chunks)

    def body(ctx_any, cls_v, out_any, buf, sem_in, sem_out):
        copies_in = {}
        copies_out = {}

        def issue_in(j):
            slot = j % NBUF
            if j >= NBUF:
                copies_out[j - NBUF].wait()
            start, sz, _ = chunks[j]
            cp = pltpu.make_async_copy(
                ctx_any.at[pl.ds(start, sz)], buf.at[slot, pl.ds(0, sz)],
                sem_in.at[slot],
            )
            cp.start()
            copies_in[j] = cp

        for j in range(min(LA, nchunk)):
            issue_in(j)

        for i in range(nchunk):
            slot = i % NBUF
            copies_in[i].wait()
            start, sz, b = chunks[i]
            buf[slot, pl.ds(0, sz)] = buf[slot, pl.ds(0, sz)] + cls_v[b]
            cp = pltpu.make_async_copy(
                buf.at[slot, pl.ds(0, sz)], out_any.at[pl.ds(start, sz)],
                sem_out.at[slot],
            )
            cp.start()
            copies_out[i] = cp
            if i + LA < nchunk:
                issue_in(i + LA)

        for i in range(max(0, nchunk - NBUF), nchunk):
            copies_out[i].wait()

    return body


@jax.jit
def kernel(ctx_vec, labels, emb_weight):
    batch, seq, d = ctx_vec.shape
    labels8 = jnp.zeros((8,), jnp.int32).at[:batch].set(labels.astype(jnp.int32))
    cls8 = _sc_gather(labels8, emb_weight)
    flat = ctx_vec.reshape(batch * seq, d)
    out = pl.pallas_call(
        _make_body(batch, seq, d),
        in_specs=[
            pl.BlockSpec(memory_space=pltpu.MemorySpace.HBM),
            pl.BlockSpec(memory_space=pltpu.VMEM),
        ],
        out_specs=pl.BlockSpec(memory_space=pltpu.MemorySpace.HBM),
        out_shape=jax.ShapeDtypeStruct((batch * seq, d), ctx_vec.dtype),
        scratch_shapes=[
            pltpu.VMEM((NBUF, CMAX, d), jnp.float32),
            pltpu.SemaphoreType.DMA((NBUF,)),
            pltpu.SemaphoreType.DMA((NBUF,)),
        ],
        compiler_params=pltpu.CompilerParams(
            vmem_limit_bytes=60 * 1024 * 1024,
        ),
    )(flat, cls8)
    return out.reshape(batch, seq, d)


# speedup vs baseline: 1.2477x; 1.2477x over previous
import jax
import jax.numpy as jnp
from jax.experimental import pallas as pl
from jax.experimental.pallas import tpu as pltpu

CMAX = 1024
NBUF = 6
LA = 4


def _chunk_schedule(batch, seq):
    first = [CMAX] * (seq // CMAX)
    mid = [CMAX] * (seq // CMAX)
    last = list(reversed(first))
    chunks = []
    for b in range(batch):
        sizes = first if b == 0 else (last if b == batch - 1 else mid)
        r = b * seq
        for sz in sizes:
            chunks.append((r, sz, b))
            r += sz
    return chunks


def _make_body(batch, seq, d):
    chunks = _chunk_schedule(batch, seq)
    nchunk = len(chunks)

    def body(labels_sm, ctx_any, emb_any, out_any, buf, cls, sem_cls, sem_in, sem_out):
        copies_in = {}
        copies_out = {}

        def issue_in(j):
            slot = j % NBUF
            if j >= NBUF:
                copies_out[j - NBUF].wait()
            start, sz, _ = chunks[j]
            cp = pltpu.make_async_copy(
                ctx_any.at[pl.ds(start, sz)], buf.at[slot, pl.ds(0, sz)],
                sem_in.at[slot],
            )
            cp.start()
            copies_in[j] = cp

        for j in range(min(LA, nchunk)):
            issue_in(j)

        cls_copies = []
        for b in range(batch):
            cp = pltpu.make_async_copy(emb_any.at[labels_sm[b]], cls.at[b], sem_cls)
            cp.start()
            cls_copies.append(cp)
        for cp in cls_copies:
            cp.wait()

        for i in range(nchunk):
            slot = i % NBUF
            copies_in[i].wait()
            start, sz, b = chunks[i]
            buf[slot, pl.ds(0, sz)] = buf[slot, pl.ds(0, sz)] + cls[b]
            cp = pltpu.make_async_copy(
                buf.at[slot, pl.ds(0, sz)], out_any.at[pl.ds(start, sz)],
                sem_out.at[slot],
            )
            cp.start()
            copies_out[i] = cp
            if i + LA < nchunk:
                issue_in(i + LA)

        for i in range(max(0, nchunk - NBUF), nchunk):
            copies_out[i].wait()

    return body


@jax.jit
def kernel(ctx_vec, labels, emb_weight):
    batch, seq, d = ctx_vec.shape
    flat = ctx_vec.reshape(batch * seq, d)
    out = pl.pallas_call(
        _make_body(batch, seq, d),
        in_specs=[
            pl.BlockSpec(memory_space=pltpu.SMEM),
            pl.BlockSpec(memory_space=pltpu.MemorySpace.HBM),
            pl.BlockSpec(memory_space=pltpu.MemorySpace.HBM),
        ],
        out_specs=pl.BlockSpec(memory_space=pltpu.MemorySpace.HBM),
        out_shape=jax.ShapeDtypeStruct((batch * seq, d), ctx_vec.dtype),
        scratch_shapes=[
            pltpu.VMEM((NBUF, CMAX, d), jnp.float32),
            pltpu.VMEM((batch, d), jnp.float32),
            pltpu.SemaphoreType.DMA,
            pltpu.SemaphoreType.DMA((NBUF,)),
            pltpu.SemaphoreType.DMA((NBUF,)),
        ],
        compiler_params=pltpu.CompilerParams(
            vmem_limit_bytes=60 * 1024 * 1024,
        ),
    )(labels.astype(jnp.int32), flat, emb_weight)
    return out.reshape(batch, seq, d)


# speedup vs baseline: 1.2505x; 1.0023x over previous
import jax
import jax.numpy as jnp
from jax.experimental import pallas as pl
from jax.experimental.pallas import tpu as pltpu

C = 1024
NBUF = 5
LA = 3
NSPLIT = 2


def _make_body(batch, seq, d):
    nchunk = (batch * seq) // C
    H = C // NSPLIT

    def body(labels_sm, ctx_any, emb_any, out_any, buf, cls, sem_cls, sem_in, sem_out):
        copies_in = {}
        copies_out = {}

        def issue_in(j):
            slot = j % NBUF
            if j >= NBUF:
                for cp in copies_out[j - NBUF]:
                    cp.wait()
            cps = []
            for h in range(NSPLIT):
                cp = pltpu.make_async_copy(
                    ctx_any.at[pl.ds(j * C + h * H, H)],
                    buf.at[slot, pl.ds(h * H, H)],
                    sem_in.at[slot, h],
                )
                cp.start()
                cps.append(cp)
            copies_in[j] = cps

        for j in range(min(LA, nchunk)):
            issue_in(j)

        cls_copies = []
        for b in range(batch):
            cp = pltpu.make_async_copy(emb_any.at[labels_sm[b]], cls.at[b], sem_cls)
            cp.start()
            cls_copies.append(cp)
        for cp in cls_copies:
            cp.wait()

        for i in range(nchunk):
            slot = i % NBUF
            for cp in copies_in[i]:
                cp.wait()
            b = (i * C) // seq
            buf[slot] = buf[slot] + cls[b]
            cps = []
            for h in range(NSPLIT):
                cp = pltpu.make_async_copy(
                    buf.at[slot, pl.ds(h * H, H)],
                    out_any.at[pl.ds(i * C + h * H, H)],
                    sem_out.at[slot, h],
                )
                cp.start()
                cps.append(cp)
            copies_out[i] = cps
            if i + LA < nchunk:
                issue_in(i + LA)

        for i in range(max(0, nchunk - NBUF), nchunk):
            for cp in copies_out[i]:
                cp.wait()

    return body


@jax.jit
def kernel(ctx_vec, labels, emb_weight):
    batch, seq, d = ctx_vec.shape
    flat = ctx_vec.reshape(batch * seq, d)
    out = pl.pallas_call(
        _make_body(batch, seq, d),
        in_specs=[
            pl.BlockSpec(memory_space=pltpu.SMEM),
            pl.BlockSpec(memory_space=pltpu.MemorySpace.HBM),
            pl.BlockSpec(memory_space=pltpu.MemorySpace.HBM),
        ],
        out_specs=pl.BlockSpec(memory_space=pltpu.MemorySpace.HBM),
        out_shape=jax.ShapeDtypeStruct((batch * seq, d), ctx_vec.dtype),
        scratch_shapes=[
            pltpu.VMEM((NBUF, C, d), jnp.float32),
            pltpu.VMEM((batch, d), jnp.float32),
            pltpu.SemaphoreType.DMA,
            pltpu.SemaphoreType.DMA((NBUF, NSPLIT)),
            pltpu.SemaphoreType.DMA((NBUF, NSPLIT)),
        ],
        compiler_params=pltpu.CompilerParams(
            vmem_limit_bytes=60 * 1024 * 1024,
        ),
    )(labels.astype(jnp.int32), flat, emb_weight)
    return out.reshape(batch, seq, d)
